# Initial kernel scaffold; baseline (speedup 1.0000x reference)
import jax
import jax.numpy as jnp
from jax.experimental import pallas as pl


def kernel(x, edge_index, edge_attr, u, batch, W1, b1, g1, be1, W2, b2, g2, be2, W3, b3):
    raise NotImplementedError("write your pallas kernel here")


# speedup vs baseline: 14.2133x; 14.2133x over previous
import functools

import jax
import jax.numpy as jnp
from jax import lax
from jax.experimental import pallas as pl
from jax.experimental.pallas import tpu as pltpu
from jax.experimental.pallas import tpu_sc as plsc

N = 10000
E = 320000
D = 128
DE = 16
G = 256

NC = 2
NS = 16
NW = NC * NS

EW = E // NW
PE = 80
NPIECE = EW // PE
ECHUNK = 2000
NCHUNK = EW // ECHUNK
PPC = ECHUNK // PE

PN = 80
NPN = N // PN


def kernel(x, edge_index, edge_attr, u, batch, W1, b1, g1, be1, W2, b2, g2, be2, W3, b3):
    col = edge_index[1]

    mesh = plsc.VectorSubcoreMesh(core_axis_name="c", subcore_axis_name="s",
                                  num_cores=NC, num_subcores=NS)

    @functools.partial(
        pl.kernel,
        out_type=(
            jax.ShapeDtypeStruct((NC, G, D), jnp.float32),
            jax.ShapeDtypeStruct((NC, G, DE), jnp.float32),
        ),
        mesh=mesh,
        compiler_params=pltpu.CompilerParams(needs_layout_passes=False,
                                             use_tc_tiling_on_sc=False),
        scratch_types=(
            pltpu.VMEM((N,), jnp.int32),
            pltpu.VMEM((EW,), jnp.int32),
            pltpu.VMEM((NPIECE, PE), jnp.int32),
            pltpu.VMEM((ECHUNK, DE), jnp.float32),
            pltpu.VMEM((PN, D), jnp.float32),
            pltpu.VMEM((1, PN), jnp.int32),
            pltpu.VMEM((16, D), jnp.float32),
            pltpu.VMEM((16, DE), jnp.float32),
            pltpu.VMEM_SHARED((G, D), jnp.float32),
            pltpu.VMEM_SHARED((G, DE), jnp.float32),
        ),
    )
    def sc_agg(x_hbm, col_hbm, batch_hbm, attr_hbm, node_out, edge_out,
               batch_v, col_v, seg_v, attr_v, x_v, nidx_v, zb_big, zb_small,
               acc_node, acc_edge):
        c = lax.axis_index("c")
        s = lax.axis_index("s")
        w = c * NS + s

        zvec = jnp.zeros((16,), jnp.float32)
        for r in range(16):
            for k in range(D // 16):
                zb_big[r, pl.ds(k * 16, 16)] = zvec
            zb_small[r, :] = zvec
        pltpu.sync_copy(zb_big, acc_node.at[pl.ds(s * 16, 16)])
        pltpu.sync_copy(zb_small, acc_edge.at[pl.ds(s * 16, 16)])

        pltpu.sync_copy(batch_hbm, batch_v)
        pltpu.sync_copy(col_hbm.at[pl.ds(w * EW, EW)], col_v)

        @pl.loop(0, NPIECE)
        def _gather(p):
            for k in range(PE // 16):
                cv = col_v[pl.ds(p * PE + k * 16, 16)]
                seg_v[p, pl.ds(k * 16, 16)] = plsc.load_gather(batch_v, [cv])

        plsc.subcore_barrier()

        @pl.loop(0, NCHUNK)
        def _echunk(cc):
            pltpu.sync_copy(attr_hbm.at[pl.ds(w * EW + cc * ECHUNK, ECHUNK)],
                            attr_v)

            @pl.loop(0, PPC)
            def _epiece(q):
                piece = cc * PPC + q
                pltpu.sync_copy(attr_v.at[pl.ds(q * PE, PE)],
                                acc_edge.at[seg_v.at[piece]], add=True)

        @pl.loop(0, (NPN - 1) // NW + 1)
        def _npiece(t):
            p = w + t * NW

            @pl.when(p < NPN)
            def _():
                pltpu.sync_copy(batch_hbm.at[pl.ds(p * PN, PN)], nidx_v.at[0])
                pltpu.sync_copy(x_hbm.at[pl.ds(p * PN, PN)], x_v)
                pltpu.sync_copy(x_v, acc_node.at[nidx_v.at[0]], add=True)

        plsc.subcore_barrier()

        pltpu.sync_copy(acc_node.at[pl.ds(s * 16, 16)],
                        node_out.at[c, pl.ds(s * 16, 16)])
        pltpu.sync_copy(acc_edge.at[pl.ds(s * 16, 16)],
                        edge_out.at[c, pl.ds(s * 16, 16)])

    node_p, edge_p = sc_agg(x, col, batch, edge_attr)

    W1a = W1[:D]
    W1b = W1[D:]

    def mlp_body(node_ref, edge_ref, W1a_ref, W1b_ref, b1_ref, g1_ref, be1_ref,
                 W2_ref, b2_ref, g2_ref, be2_ref, W3_ref, b3_ref, out_ref):
        na = node_ref[0] + node_ref[1]
        ea = edge_ref[0] + edge_ref[1]
        h = (jnp.dot(na, W1a_ref[...], preferred_element_type=jnp.float32)
             + jnp.dot(ea, W1b_ref[...], preferred_element_type=jnp.float32)
             + b1_ref[...])

        def act_bn(h, gamma, beta):
            h = jnp.where(h >= 0, h, 0.01 * h)
            mean = jnp.mean(h, axis=0, keepdims=True)
            var = jnp.mean((h - mean) ** 2, axis=0, keepdims=True)
            return (h - mean) / jnp.sqrt(var + 1e-5) * gamma + beta

        h = act_bn(h, g1_ref[...], be1_ref[...])
        h = jnp.dot(h, W2_ref[...], preferred_element_type=jnp.float32) + b2_ref[...]
        h = act_bn(h, g2_ref[...], be2_ref[...])
        out_ref[...] = (jnp.dot(h, W3_ref[...], preferred_element_type=jnp.float32)
                        + b3_ref[...])

    out = pl.pallas_call(
        mlp_body,
        out_shape=jax.ShapeDtypeStruct((G, D), jnp.float32),
    )(node_p, edge_p, W1a, W1b, b1.reshape(1, -1), g1.reshape(1, -1),
      be1.reshape(1, -1), W2, b2.reshape(1, -1), g2.reshape(1, -1),
      be2.reshape(1, -1), W3, b3.reshape(1, -1))
    return out


# speedup vs baseline: 16.8780x; 1.1875x over previous
import functools

import jax
import jax.numpy as jnp
from jax import lax
from jax.experimental import pallas as pl
from jax.experimental.pallas import tpu as pltpu
from jax.experimental.pallas import tpu_sc as plsc

N = 10000
E = 320000
D = 128
DE = 16
G = 256

NC = 2
NS = 16
NW = NC * NS

LT = E // 128
TPW = LT // NW
EPW = TPW * 128
XTRA = LT - TPW * NW
XBASE = TPW * NW

CH = 8
NCH = TPW // CH
TAILT = TPW - NCH * CH
ECH = CH * 128

NB = 16
SEGW = G * DE

PN = 80
NPN = N // PN


def kernel(x, edge_index, edge_attr, u, batch, W1, b1, g1, be1, W2, b2, g2, be2, W3, b3):
    ea_t = edge_attr.T

    mesh = plsc.VectorSubcoreMesh(core_axis_name="c", subcore_axis_name="s",
                                  num_cores=NC, num_subcores=NS)

    @functools.partial(
        pl.kernel,
        out_type=(
            jax.ShapeDtypeStruct((NC, G, D), jnp.float32),
            jax.ShapeDtypeStruct((NC, G * DE), jnp.float32),
        ),
        mesh=mesh,
        compiler_params=pltpu.CompilerParams(needs_layout_passes=False),
        scratch_types=(
            pltpu.VMEM((N,), jnp.int32),
            pltpu.VMEM((EPW + 128,), jnp.int32),
            pltpu.VMEM((DE, ECH), jnp.float32),
            pltpu.VMEM((NB * SEGW,), jnp.float32),
            pltpu.VMEM((NB, G), jnp.float32),
            pltpu.VMEM((G,), jnp.float32),
            pltpu.VMEM((PN, D), jnp.float32),
            pltpu.VMEM((1, PN), jnp.int32),
            pltpu.VMEM((16, D), jnp.float32),
            pltpu.VMEM_SHARED((G, D), jnp.float32),
            pltpu.VMEM_SHARED((NB, SEGW), jnp.float32),
        ),
    )
    def sc_agg(x_hbm, ei_hbm, eat_hbm, batch_hbm, node_out, edge_out,
               batch_v, cs_v, stage_v, bank_v, tmp_v, out_v,
               x_v, nidx_v, zb_v, acc_node, slab):
        c = lax.axis_index("c")
        s = lax.axis_index("s")
        w = c * NS + s
        t0 = w * TPW
        e_base = t0 * 128

        zvec = jnp.zeros((16,), jnp.float32)
        lane_off = lax.iota(jnp.int32, 16) * SEGW

        for r in range(16):
            for k in range(D // 16):
                zb_v[r, pl.ds(k * 16, 16)] = zvec
        pltpu.sync_copy(zb_v, acc_node.at[pl.ds(s * 16, 16)])

        @pl.loop(0, NB * SEGW // 64, unroll=4)
        def _zero(i):
            for k in range(4):
                bank_v[pl.ds(i * 64 + k * 16, 16)] = zvec

        pltpu.sync_copy(batch_hbm, batch_v)
        pltpu.sync_copy(ei_hbm.at[1, pl.ds(e_base, EPW)], cs_v.at[pl.ds(0, EPW)])

        @pl.when(w < XTRA)
        def _():
            pltpu.sync_copy(ei_hbm.at[1, pl.ds((XBASE + w) * 128, 128)],
                            cs_v.at[pl.ds(EPW, 128)])

        @pl.loop(0, EPW // 16)
        def _gather(g):
            cv = cs_v[pl.ds(g * 16, 16)]
            cs_v[pl.ds(g * 16, 16)] = plsc.load_gather(batch_v, [cv])

        @pl.when(w < XTRA)
        def _():
            for k in range(128 // 16):
                cv = cs_v[pl.ds(EPW + k * 16, 16)]
                cs_v[pl.ds(EPW + k * 16, 16)] = plsc.load_gather(batch_v, [cv])

        plsc.subcore_barrier()

        def scatter_groups(local_e0, ngroups):
            @pl.loop(0, ngroups)
            def _(g):
                seg16 = cs_v[pl.ds(local_e0 + g * 16, 16)]
                base = seg16 * DE + lane_off
                for d in range(DE):
                    v = stage_v[d, pl.ds(g * 16, 16)]
                    plsc.addupdate_scatter(bank_v, [base + d], v)

        for cc in range(NCH):
            pltpu.sync_copy(eat_hbm.at[:, pl.ds(e_base + cc * ECH, ECH)],
                            stage_v)
            scatter_groups(cc * ECH, ECH // 16)

        tail_e = NCH * ECH
        pltpu.sync_copy(eat_hbm.at[:, pl.ds(e_base + tail_e, TAILT * 128)],
                        stage_v.at[:, pl.ds(0, TAILT * 128)])
        scatter_groups(tail_e, TAILT * 128 // 16)

        @pl.when(w < XTRA)
        def _():
            pltpu.sync_copy(eat_hbm.at[:, pl.ds((XBASE + w) * 128, 128)],
                            stage_v.at[:, pl.ds(0, 128)])

            @pl.loop(0, 128 // 16)
            def _(g):
                seg16 = cs_v[pl.ds(EPW + g * 16, 16)]
                base = seg16 * DE + lane_off
                for d in range(DE):
                    v = stage_v[d, pl.ds(g * 16, 16)]
                    plsc.addupdate_scatter(bank_v, [base + d], v)

        @pl.loop(0, SEGW // 16)
        def _bankred(i):
            acc = bank_v[pl.ds(i * 16, 16)]
            for b in range(1, NB):
                acc = acc + bank_v[pl.ds(b * SEGW + i * 16, 16)]
            bank_v[pl.ds(i * 16, 16)] = acc

        @pl.loop(0, (NPN - 1) // NW + 1)
        def _npiece(t):
            p = w + t * NW

            @pl.when(p < NPN)
            def _():
                pltpu.sync_copy(batch_hbm.at[pl.ds(p * PN, PN)], nidx_v.at[0])
                pltpu.sync_copy(x_hbm.at[pl.ds(p * PN, PN)], x_v)
                pltpu.sync_copy(x_v, acc_node.at[nidx_v.at[0]], add=True)

        pltpu.sync_copy(bank_v.at[pl.ds(0, SEGW)], slab.at[s])
        plsc.subcore_barrier()
        pltpu.sync_copy(slab.at[:, pl.ds(s * G, G)], tmp_v)
        for i in range(G // 16):
            acc = zvec
            for b in range(NB):
                acc = acc + tmp_v[b, pl.ds(i * 16, 16)]
            out_v[pl.ds(i * 16, 16)] = acc
        pltpu.sync_copy(out_v, edge_out.at[c, pl.ds(s * G, G)])

        pltpu.sync_copy(acc_node.at[pl.ds(s * 16, 16)],
                        node_out.at[c, pl.ds(s * 16, 16)])

    node_p, edge_p = sc_agg(x, edge_index, ea_t, batch)
    edge_p = edge_p.reshape(NC, G, DE)

    W1a = W1[:D]
    W1b = W1[D:]

    def mlp_body(node_ref, edge_ref, W1a_ref, W1b_ref, b1_ref, g1_ref, be1_ref,
                 W2_ref, b2_ref, g2_ref, be2_ref, W3_ref, b3_ref, out_ref):
        na = node_ref[0] + node_ref[1]
        ea = edge_ref[0] + edge_ref[1]
        h = (jnp.dot(na, W1a_ref[...], preferred_element_type=jnp.float32)
             + jnp.dot(ea, W1b_ref[...], preferred_element_type=jnp.float32)
             + b1_ref[...])

        def act_bn(h, gamma, beta):
            h = jnp.where(h >= 0, h, 0.01 * h)
            mean = jnp.mean(h, axis=0, keepdims=True)
            var = jnp.mean((h - mean) ** 2, axis=0, keepdims=True)
            return (h - mean) / jnp.sqrt(var + 1e-5) * gamma + beta

        h = act_bn(h, g1_ref[...], be1_ref[...])
        h = jnp.dot(h, W2_ref[...], preferred_element_type=jnp.float32) + b2_ref[...]
        h = act_bn(h, g2_ref[...], be2_ref[...])
        out_ref[...] = (jnp.dot(h, W3_ref[...], preferred_element_type=jnp.float32)
                        + b3_ref[...])

    out = pl.pallas_call(
        mlp_body,
        out_shape=jax.ShapeDtypeStruct((G, D), jnp.float32),
    )(node_p, edge_p, W1a, W1b, b1.reshape(1, -1), g1.reshape(1, -1),
      be1.reshape(1, -1), W2, b2.reshape(1, -1), g2.reshape(1, -1),
      be2.reshape(1, -1), W3, b3.reshape(1, -1))
    return out


# speedup vs baseline: 24.9320x; 1.4772x over previous
import functools

import jax
import jax.numpy as jnp
from jax import lax
from jax.experimental import pallas as pl
from jax.experimental.pallas import tpu as pltpu
from jax.experimental.pallas import tpu_sc as plsc

N = 10000
E = 320000
D = 128
DE = 16
G = 256

NC = 2
NS = 16
NW = NC * NS

LT = E // 128
TPW = LT // NW
EPW = TPW * 128
XTRA = LT - TPW * NW
XBASE = TPW * NW

CH = 8
NCH = TPW // CH
TAILT = TPW - NCH * CH
ECH = CH * 128

NB = 16
SEGW = G * DE

PN = 80
NPN = N // PN


def kernel(x, edge_index, edge_attr, u, batch, W1, b1, g1, be1, W2, b2, g2, be2, W3, b3):
    ea_t = edge_attr.T

    mesh = plsc.VectorSubcoreMesh(core_axis_name="c", subcore_axis_name="s",
                                  num_cores=NC, num_subcores=NS)

    @functools.partial(
        pl.kernel,
        out_type=(
            jax.ShapeDtypeStruct((NC, G, D), jnp.float32),
            jax.ShapeDtypeStruct((NC, G * DE), jnp.float32),
        ),
        mesh=mesh,
        compiler_params=pltpu.CompilerParams(needs_layout_passes=False),
        scratch_types=(
            pltpu.VMEM((N,), jnp.int32),
            pltpu.VMEM((EPW + 128,), jnp.int32),
            pltpu.VMEM((DE, ECH), jnp.float32),
            pltpu.VMEM((NB * SEGW,), jnp.float32),
            pltpu.VMEM((NB, G), jnp.float32),
            pltpu.VMEM((G,), jnp.float32),
            pltpu.VMEM((PN, D), jnp.float32),
            pltpu.VMEM((1, PN), jnp.int32),
            pltpu.VMEM((16, D), jnp.float32),
            pltpu.VMEM_SHARED((G, D), jnp.float32),
            pltpu.VMEM_SHARED((NB, SEGW), jnp.float32),
        ),
    )
    def sc_agg(x_hbm, ei_hbm, eat_hbm, batch_hbm, node_out, edge_out,
               batch_v, cs_v, stage_v, bank_v, tmp_v, out_v,
               x_v, nidx_v, zb_v, acc_node, slab):
        c = lax.axis_index("c")
        s = lax.axis_index("s")
        w = c * NS + s
        t0 = w * TPW
        e_base = t0 * 128

        zvec = jnp.zeros((16,), jnp.float32)
        lane_iota = lax.iota(jnp.int32, 16)
        lane_off = lane_iota * SEGW
        rots = [(lane_iota + d) & 15 for d in range(DE)]

        for r in range(16):
            for k in range(D // 16):
                zb_v[r, pl.ds(k * 16, 16)] = zvec
        pltpu.sync_copy(zb_v, acc_node.at[pl.ds(s * 16, 16)])

        @pl.loop(0, NB * SEGW // 64, unroll=4)
        def _zero(i):
            for k in range(4):
                bank_v[pl.ds(i * 64 + k * 16, 16)] = zvec

        pltpu.sync_copy(batch_hbm, batch_v)
        pltpu.sync_copy(ei_hbm.at[1, pl.ds(e_base, EPW)], cs_v.at[pl.ds(0, EPW)])

        @pl.when(w < XTRA)
        def _():
            pltpu.sync_copy(ei_hbm.at[1, pl.ds((XBASE + w) * 128, 128)],
                            cs_v.at[pl.ds(EPW, 128)])

        @pl.loop(0, EPW // 16)
        def _gather(g):
            cv = cs_v[pl.ds(g * 16, 16)]
            cs_v[pl.ds(g * 16, 16)] = plsc.load_gather(batch_v, [cv])

        @pl.when(w < XTRA)
        def _():
            for k in range(128 // 16):
                cv = cs_v[pl.ds(EPW + k * 16, 16)]
                cs_v[pl.ds(EPW + k * 16, 16)] = plsc.load_gather(batch_v, [cv])

        plsc.subcore_barrier()

        def scatter_groups(local_e0, ngroups):
            @pl.loop(0, ngroups)
            def _(g):
                seg16 = cs_v[pl.ds(local_e0 + g * 16, 16)]
                base = seg16 * DE + lane_off
                for d in range(DE):
                    v = stage_v[d, pl.ds(g * 16, 16)]
                    plsc.addupdate_scatter(bank_v, [base + rots[d]], v)

        for cc in range(NCH):
            pltpu.sync_copy(eat_hbm.at[:, pl.ds(e_base + cc * ECH, ECH)],
                            stage_v)
            scatter_groups(cc * ECH, ECH // 16)

        tail_e = NCH * ECH
        pltpu.sync_copy(eat_hbm.at[:, pl.ds(e_base + tail_e, TAILT * 128)],
                        stage_v.at[:, pl.ds(0, TAILT * 128)])
        scatter_groups(tail_e, TAILT * 128 // 16)

        @pl.when(w < XTRA)
        def _():
            pltpu.sync_copy(eat_hbm.at[:, pl.ds((XBASE + w) * 128, 128)],
                            stage_v.at[:, pl.ds(0, 128)])
            scatter_groups(EPW, 128 // 16)

        @pl.loop(0, SEGW // 16)
        def _bankred(i):
            acc = bank_v[pl.ds(i * 16, 16)]
            for b in range(1, NB):
                idx = rots[b] + (b * SEGW + i * 16)
                acc = acc + plsc.load_gather(bank_v, [idx])
            bank_v[pl.ds(i * 16, 16)] = acc

        @pl.loop(0, (NPN - 1) // NW + 1)
        def _npiece(t):
            p = w + t * NW

            @pl.when(p < NPN)
            def _():
                pltpu.sync_copy(batch_hbm.at[pl.ds(p * PN, PN)], nidx_v.at[0])
                pltpu.sync_copy(x_hbm.at[pl.ds(p * PN, PN)], x_v)
                pltpu.sync_copy(x_v, acc_node.at[nidx_v.at[0]], add=True)

        pltpu.sync_copy(bank_v.at[pl.ds(0, SEGW)], slab.at[s])
        plsc.subcore_barrier()
        pltpu.sync_copy(slab.at[:, pl.ds(s * G, G)], tmp_v)
        for i in range(G // 16):
            acc = zvec
            for b in range(NB):
                acc = acc + tmp_v[b, pl.ds(i * 16, 16)]
            out_v[pl.ds(i * 16, 16)] = acc
        pltpu.sync_copy(out_v, edge_out.at[c, pl.ds(s * G, G)])

        pltpu.sync_copy(acc_node.at[pl.ds(s * 16, 16)],
                        node_out.at[c, pl.ds(s * 16, 16)])

    node_p, edge_p = sc_agg(x, edge_index, ea_t, batch)
    edge_p = edge_p.reshape(NC, G, DE)

    W1a = W1[:D]
    W1b = W1[D:]

    def mlp_body(node_ref, edge_ref, W1a_ref, W1b_ref, b1_ref, g1_ref, be1_ref,
                 W2_ref, b2_ref, g2_ref, be2_ref, W3_ref, b3_ref, out_ref):
        na = node_ref[0] + node_ref[1]
        ea = edge_ref[0] + edge_ref[1]
        h = (jnp.dot(na, W1a_ref[...], preferred_element_type=jnp.float32)
             + jnp.dot(ea, W1b_ref[...], preferred_element_type=jnp.float32)
             + b1_ref[...])

        def act_bn(h, gamma, beta):
            h = jnp.where(h >= 0, h, 0.01 * h)
            mean = jnp.mean(h, axis=0, keepdims=True)
            var = jnp.mean((h - mean) ** 2, axis=0, keepdims=True)
            return (h - mean) / jnp.sqrt(var + 1e-5) * gamma + beta

        h = act_bn(h, g1_ref[...], be1_ref[...])
        h = jnp.dot(h, W2_ref[...], preferred_element_type=jnp.float32) + b2_ref[...]
        h = act_bn(h, g2_ref[...], be2_ref[...])
        out_ref[...] = (jnp.dot(h, W3_ref[...], preferred_element_type=jnp.float32)
                        + b3_ref[...])

    out = pl.pallas_call(
        mlp_body,
        out_shape=jax.ShapeDtypeStruct((G, D), jnp.float32),
    )(node_p, edge_p, W1a, W1b, b1.reshape(1, -1), g1.reshape(1, -1),
      be1.reshape(1, -1), W2, b2.reshape(1, -1), g2.reshape(1, -1),
      be2.reshape(1, -1), W3, b3.reshape(1, -1))
    return out


# speedup vs baseline: 34.3910x; 1.3794x over previous
import functools

import jax
import jax.numpy as jnp
from jax import lax
from jax.experimental import pallas as pl
from jax.experimental.pallas import tpu as pltpu
from jax.experimental.pallas import tpu_sc as plsc

N = 10000
E = 320000
D = 128
DE = 16
G = 256

NC = 2
NS = 16
NW = NC * NS

LT = E // 128
TPW = LT // NW
EPW = TPW * 128
XTRA = LT - TPW * NW
XBASE = TPW * NW

CH = 8
NCH = TPW // CH
TAILT = TPW - NCH * CH
ECH = CH * 128

NB = 16
SEGW = G * DE

PN = 80
NPN = N // PN


def kernel(x, edge_index, edge_attr, u, batch, W1, b1, g1, be1, W2, b2, g2, be2, W3, b3):
    ea_t = edge_attr.T

    mesh = plsc.VectorSubcoreMesh(core_axis_name="c", subcore_axis_name="s",
                                  num_cores=NC, num_subcores=NS)

    @functools.partial(
        pl.kernel,
        out_type=(
            jax.ShapeDtypeStruct((NC, G, D), jnp.float32),
            jax.ShapeDtypeStruct((NC, G * DE), jnp.float32),
        ),
        mesh=mesh,
        compiler_params=pltpu.CompilerParams(needs_layout_passes=False),
        scratch_types=(
            pltpu.VMEM((N,), jnp.int32),
            pltpu.VMEM((EPW + 128,), jnp.int32),
            pltpu.VMEM((DE, ECH), jnp.float32),
            pltpu.VMEM((NB * SEGW,), jnp.float32),
            pltpu.VMEM((NB, G), jnp.float32),
            pltpu.VMEM((G,), jnp.float32),
            pltpu.VMEM((PN, D), jnp.float32),
            pltpu.VMEM((1, PN), jnp.int32),
            pltpu.VMEM((16, D), jnp.float32),
            pltpu.VMEM_SHARED((G, D), jnp.float32),
            pltpu.VMEM_SHARED((NB, SEGW), jnp.float32),
        ),
    )
    def sc_agg(x_hbm, ei_hbm, eat_hbm, batch_hbm, node_out, edge_out,
               batch_v, cs_v, stage_v, bank_v, tmp_v, out_v,
               x_v, nidx_v, zb_v, acc_node, slab):
        c = lax.axis_index("c")
        s = lax.axis_index("s")
        w = c * NS + s
        t0 = w * TPW
        e_base = t0 * 128

        zvec = jnp.zeros((16,), jnp.float32)
        lane_iota = lax.iota(jnp.int32, 16)
        lane_off = lane_iota * SEGW
        rots = [(lane_iota + d) & 15 for d in range(DE)]

        for r in range(16):
            for k in range(D // 16):
                zb_v[r, pl.ds(k * 16, 16)] = zvec
        pltpu.sync_copy(zb_v, acc_node.at[pl.ds(s * 16, 16)])

        @pl.loop(0, NB * SEGW // 64, unroll=4)
        def _zero(i):
            for k in range(4):
                bank_v[pl.ds(i * 64 + k * 16, 16)] = zvec

        pltpu.sync_copy(batch_hbm, batch_v)
        pltpu.sync_copy(ei_hbm.at[1, pl.ds(e_base, EPW)], cs_v.at[pl.ds(0, EPW)])

        @pl.when(w < XTRA)
        def _():
            pltpu.sync_copy(ei_hbm.at[1, pl.ds((XBASE + w) * 128, 128)],
                            cs_v.at[pl.ds(EPW, 128)])

        @pl.loop(0, EPW // 16, unroll=4)
        def _gather(g):
            cv = cs_v[pl.ds(g * 16, 16)]
            cs_v[pl.ds(g * 16, 16)] = plsc.load_gather(batch_v, [cv])

        @pl.when(w < XTRA)
        def _():
            for k in range(128 // 16):
                cv = cs_v[pl.ds(EPW + k * 16, 16)]
                cs_v[pl.ds(EPW + k * 16, 16)] = plsc.load_gather(batch_v, [cv])

        plsc.subcore_barrier()

        def scatter_groups(local_e0, ngroups):
            @pl.loop(0, ngroups)
            def _(g):
                seg16 = cs_v[pl.ds(local_e0 + g * 16, 16)]
                base = seg16 * DE + lane_off
                vals = [stage_v[d, pl.ds(g * 16, 16)] for d in range(DE)]
                idxs = [base + rots[d] for d in range(DE)]
                for d in range(DE):
                    plsc.addupdate_scatter(bank_v, [idxs[d]], vals[d])

        for cc in range(NCH):
            pltpu.sync_copy(eat_hbm.at[:, pl.ds(e_base + cc * ECH, ECH)],
                            stage_v)
            scatter_groups(cc * ECH, ECH // 16)

        tail_e = NCH * ECH
        pltpu.sync_copy(eat_hbm.at[:, pl.ds(e_base + tail_e, TAILT * 128)],
                        stage_v.at[:, pl.ds(0, TAILT * 128)])
        scatter_groups(tail_e, TAILT * 128 // 16)

        @pl.when(w < XTRA)
        def _():
            pltpu.sync_copy(eat_hbm.at[:, pl.ds((XBASE + w) * 128, 128)],
                            stage_v.at[:, pl.ds(0, 128)])
            scatter_groups(EPW, 128 // 16)

        @pl.loop(0, SEGW // 16)
        def _bankred(i):
            acc = bank_v[pl.ds(i * 16, 16)]
            for b in range(1, NB):
                idx = rots[b] + (b * SEGW + i * 16)
                acc = acc + plsc.load_gather(bank_v, [idx])
            bank_v[pl.ds(i * 16, 16)] = acc

        @pl.loop(0, (NPN - 1) // NW + 1)
        def _npiece(t):
            p = w + t * NW

            @pl.when(p < NPN)
            def _():
                pltpu.sync_copy(batch_hbm.at[pl.ds(p * PN, PN)], nidx_v.at[0])
                pltpu.sync_copy(x_hbm.at[pl.ds(p * PN, PN)], x_v)
                pltpu.sync_copy(x_v, acc_node.at[nidx_v.at[0]], add=True)

        pltpu.sync_copy(bank_v.at[pl.ds(0, SEGW)], slab.at[s])
        plsc.subcore_barrier()
        pltpu.sync_copy(slab.at[:, pl.ds(s * G, G)], tmp_v)
        for i in range(G // 16):
            acc = zvec
            for b in range(NB):
                acc = acc + tmp_v[b, pl.ds(i * 16, 16)]
            out_v[pl.ds(i * 16, 16)] = acc
        pltpu.sync_copy(out_v, edge_out.at[c, pl.ds(s * G, G)])

        pltpu.sync_copy(acc_node.at[pl.ds(s * 16, 16)],
                        node_out.at[c, pl.ds(s * 16, 16)])

    node_p, edge_p = sc_agg(x, edge_index, ea_t, batch)
    edge_p = edge_p.reshape(NC, G, DE)

    W1a = W1[:D]
    W1b = W1[D:]

    def mlp_body(node_ref, edge_ref, W1a_ref, W1b_ref, b1_ref, g1_ref, be1_ref,
                 W2_ref, b2_ref, g2_ref, be2_ref, W3_ref, b3_ref, out_ref):
        na = node_ref[0] + node_ref[1]
        ea = edge_ref[0] + edge_ref[1]
        h = (jnp.dot(na, W1a_ref[...], preferred_element_type=jnp.float32)
             + jnp.dot(ea, W1b_ref[...], preferred_element_type=jnp.float32)
             + b1_ref[...])

        def act_bn(h, gamma, beta):
            h = jnp.where(h >= 0, h, 0.01 * h)
            mean = jnp.mean(h, axis=0, keepdims=True)
            var = jnp.mean((h - mean) ** 2, axis=0, keepdims=True)
            return (h - mean) / jnp.sqrt(var + 1e-5) * gamma + beta

        h = act_bn(h, g1_ref[...], be1_ref[...])
        h = jnp.dot(h, W2_ref[...], preferred_element_type=jnp.float32) + b2_ref[...]
        h = act_bn(h, g2_ref[...], be2_ref[...])
        out_ref[...] = (jnp.dot(h, W3_ref[...], preferred_element_type=jnp.float32)
                        + b3_ref[...])

    out = pl.pallas_call(
        mlp_body,
        out_shape=jax.ShapeDtypeStruct((G, D), jnp.float32),
    )(node_p, edge_p, W1a, W1b, b1.reshape(1, -1), g1.reshape(1, -1),
      be1.reshape(1, -1), W2, b2.reshape(1, -1), g2.reshape(1, -1),
      be2.reshape(1, -1), W3, b3.reshape(1, -1))
    return out


# speedup vs baseline: 40.7211x; 1.1841x over previous
import functools

import jax
import jax.numpy as jnp
from jax import lax
from jax.experimental import pallas as pl
from jax.experimental.pallas import tpu as pltpu
from jax.experimental.pallas import tpu_sc as plsc

N = 10000
E = 320000
D = 128
DE = 16
G = 256

NC = 2
NS = 16
NW = NC * NS

LT = E // 128
TPW = LT // NW
EPW = TPW * 128
XTRA = LT - TPW * NW
XBASE = TPW * NW

CH = 3
NCH = TPW // CH
ECH = CH * 128

NB = 16
SEGW = G * DE

PN = 40
NPN = N // PN


def kernel(x, edge_index, edge_attr, u, batch, W1, b1, g1, be1, W2, b2, g2, be2, W3, b3):
    ea_t = edge_attr.T

    mesh = plsc.VectorSubcoreMesh(core_axis_name="c", subcore_axis_name="s",
                                  num_cores=NC, num_subcores=NS)

    @functools.partial(
        pl.kernel,
        out_type=(
            jax.ShapeDtypeStruct((NC, G, D), jnp.float32),
            jax.ShapeDtypeStruct((NC, G, DE), jnp.float32),
        ),
        mesh=mesh,
        compiler_params=pltpu.CompilerParams(needs_layout_passes=False),
        scratch_types=(
            pltpu.VMEM((N,), jnp.int32),
            pltpu.VMEM((EPW + 128,), jnp.int32),
            pltpu.VMEM((DE, ECH), jnp.float32),
            pltpu.VMEM((DE, ECH), jnp.float32),
            pltpu.VMEM((NB * SEGW,), jnp.float32),
            pltpu.VMEM((DE, DE), jnp.float32),
            pltpu.VMEM((PN, D), jnp.float32),
            pltpu.VMEM((PN, D), jnp.float32),
            pltpu.VMEM((2, PN), jnp.int32),
            pltpu.VMEM_SHARED((G, D), jnp.float32),
            pltpu.VMEM_SHARED((NB, SEGW), jnp.float32),
            pltpu.SemaphoreType.DMA,
            pltpu.SemaphoreType.DMA,
            pltpu.SemaphoreType.DMA,
            pltpu.SemaphoreType.DMA,
            pltpu.SemaphoreType.DMA,
        ),
    )
    def sc_agg(x_hbm, ei_hbm, eat_hbm, batch_hbm, node_out, edge_out,
               batch_v, cs_v, stage0_v, stage1_v, bank_v, out_v,
               x0_v, x1_v, nidx_v, acc_node, slab,
               sem_b, sem_e0, sem_e1, sem_n0, sem_n1):
        c = lax.axis_index("c")
        s = lax.axis_index("s")
        w = c * NS + s
        t0 = w * TPW
        e_base = t0 * 128

        zvec = jnp.zeros((16,), jnp.float32)
        lane_iota = lax.iota(jnp.int32, 16)
        lane_off = lane_iota * SEGW
        rots = [(lane_iota + d) & 15 for d in range(DE)]
        stages = [stage0_v, stage1_v]
        sems_e = [sem_e0, sem_e1]
        xbufs = [x0_v, x1_v]
        sems_n = [sem_n0, sem_n1]

        h_batch = pltpu.async_copy(batch_hbm, batch_v, sem_b)
        h_col = pltpu.async_copy(ei_hbm.at[1, pl.ds(e_base, EPW)],
                                 cs_v.at[pl.ds(0, EPW)], sem_b)

        @pl.when(w < XTRA)
        def _():
            pltpu.async_copy(ei_hbm.at[1, pl.ds((XBASE + w) * 128, 128)],
                             cs_v.at[pl.ds(EPW, 128)], sem_b)

        h_e = pltpu.async_copy(eat_hbm.at[:, pl.ds(e_base, ECH)], stage0_v,
                               sem_e0)

        for r in range(16):
            for k in range(D // 16):
                x0_v[r, pl.ds(k * 16, 16)] = zvec
        pltpu.sync_copy(x0_v.at[pl.ds(0, 16)], acc_node.at[pl.ds(s * 16, 16)])

        @pl.loop(0, NB * SEGW // 64, unroll=4)
        def _zero(i):
            for k in range(4):
                bank_v[pl.ds(i * 64 + k * 16, 16)] = zvec

        h_batch.wait()
        h_col.wait()

        @pl.when(w < XTRA)
        def _():
            pltpu.make_async_copy(ei_hbm.at[1, pl.ds(0, 128)],
                                  cs_v.at[pl.ds(EPW, 128)], sem_b).wait()

        @pl.loop(0, EPW // 16, unroll=4)
        def _gather(g):
            cv = cs_v[pl.ds(g * 16, 16)]
            cs_v[pl.ds(g * 16, 16)] = plsc.load_gather(batch_v, [cv])

        @pl.when(w < XTRA)
        def _():
            for k in range(128 // 16):
                cv = cs_v[pl.ds(EPW + k * 16, 16)]
                cs_v[pl.ds(EPW + k * 16, 16)] = plsc.load_gather(batch_v, [cv])

        plsc.subcore_barrier()

        def scatter_groups(buf, local_e0, ngroups):
            @pl.loop(0, ngroups)
            def _(g):
                seg16 = cs_v[pl.ds(local_e0 + g * 16, 16)]
                base = seg16 * DE + lane_off
                vals = [buf[d, pl.ds(g * 16, 16)] for d in range(DE)]
                idxs = [base + rots[d] for d in range(DE)]
                for d in range(DE):
                    plsc.addupdate_scatter(bank_v, [idxs[d]], vals[d])

        h_cur = h_e
        for cc in range(NCH):
            if cc + 1 < NCH:
                h_next = pltpu.async_copy(
                    eat_hbm.at[:, pl.ds(e_base + (cc + 1) * ECH, ECH)],
                    stages[(cc + 1) % 2], sems_e[(cc + 1) % 2])
            h_cur.wait()
            scatter_groups(stages[cc % 2], cc * ECH, ECH // 16)
            if cc + 1 < NCH:
                h_cur = h_next

        @pl.when(w < XTRA)
        def _():
            pltpu.sync_copy(eat_hbm.at[:, pl.ds((XBASE + w) * 128, 128)],
                            stage0_v.at[:, pl.ds(0, 128)])
            scatter_groups(stage0_v, EPW, 128 // 16)

        def prefetch(t):
            par = t % 2
            p = w + t * NW
            hi = pltpu.async_copy(batch_hbm.at[pl.ds(p * PN, PN)],
                                  nidx_v.at[par], sems_n[par])
            hx = pltpu.async_copy(x_hbm.at[pl.ds(p * PN, PN)], xbufs[par],
                                  sems_n[par])
            return hi, hx

        handles = [None, None]
        handles[0] = prefetch(0)
        for t in range(8):
            par = t % 2
            valid = t < 7
            if t + 1 < 7:
                handles[(t + 1) % 2] = prefetch(t + 1)
            elif t + 1 == 7:
                @pl.when(w + 7 * NW < NPN)
                def _():
                    par2 = (t + 1) % 2
                    pltpu.async_copy(batch_hbm.at[pl.ds((w + 7 * NW) * PN, PN)],
                                     nidx_v.at[par2], sems_n[par2])
                    pltpu.async_copy(x_hbm.at[pl.ds((w + 7 * NW) * PN, PN)],
                                     xbufs[par2], sems_n[par2])
            if valid:
                hi, hx = handles[par]
                hi.wait()
                hx.wait()
                pltpu.sync_copy(xbufs[par], acc_node.at[nidx_v.at[par]],
                                add=True)
            else:
                @pl.when(w + 7 * NW < NPN)
                def _():
                    pltpu.make_async_copy(batch_hbm.at[pl.ds(0, PN)],
                                          nidx_v.at[par], sems_n[par]).wait()
                    pltpu.make_async_copy(x_hbm.at[pl.ds(0, PN)], xbufs[par],
                                          sems_n[par]).wait()
                    pltpu.sync_copy(xbufs[par], acc_node.at[nidx_v.at[par]],
                                    add=True)

        @pl.loop(0, SEGW // 16)
        def _bankred(i):
            acc = bank_v[pl.ds(i * 16, 16)]
            for b in range(1, NB):
                idx = rots[b] + (b * SEGW + i * 16)
                acc = acc + plsc.load_gather(bank_v, [idx])
            bank_v[pl.ds(i * 16, 16)] = acc

        pltpu.sync_copy(bank_v.at[pl.ds(0, SEGW)], slab.at[s])
        plsc.subcore_barrier()
        pltpu.sync_copy(slab.at[:, pl.ds(s * G, G)],
                        stage0_v.at[:, pl.ds(0, G)])
        for i in range(DE):
            acc = zvec
            for b in range(NB):
                acc = acc + stage0_v[b, pl.ds(i * 16, 16)]
            out_v[i, :] = acc
        pltpu.sync_copy(out_v, edge_out.at[c, pl.ds(s * DE, DE), :])

        pltpu.sync_copy(acc_node.at[pl.ds(s * 16, 16)],
                        node_out.at[c, pl.ds(s * 16, 16)])

    node_p, edge_p = sc_agg(x, edge_index, ea_t, batch)

    def mlp_body(node_ref, edge_ref, W1_ref, b1_ref, g1_ref, be1_ref,
                 W2_ref, b2_ref, g2_ref, be2_ref, W3_ref, b3_ref, out_ref):
        na = node_ref[0] + node_ref[1]
        ea = edge_ref[0] + edge_ref[1]
        h = (jnp.dot(na, W1_ref[:D, :], preferred_element_type=jnp.float32)
             + jnp.dot(ea, W1_ref[D:, :], preferred_element_type=jnp.float32)
             + b1_ref[...])

        def act_bn(h, gamma, beta):
            h = jnp.where(h >= 0, h, 0.01 * h)
            mean = jnp.mean(h, axis=0, keepdims=True)
            var = jnp.mean((h - mean) ** 2, axis=0, keepdims=True)
            return (h - mean) / jnp.sqrt(var + 1e-5) * gamma + beta

        h = act_bn(h, g1_ref[...], be1_ref[...])
        h = jnp.dot(h, W2_ref[...], preferred_element_type=jnp.float32) + b2_ref[...]
        h = act_bn(h, g2_ref[...], be2_ref[...])
        out_ref[...] = (jnp.dot(h, W3_ref[...], preferred_element_type=jnp.float32)
                        + b3_ref[...])

    out = pl.pallas_call(
        mlp_body,
        out_shape=jax.ShapeDtypeStruct((G, D), jnp.float32),
    )(node_p, edge_p, W1, b1.reshape(1, -1), g1.reshape(1, -1),
      be1.reshape(1, -1), W2, b2.reshape(1, -1), g2.reshape(1, -1),
      be2.reshape(1, -1), W3, b3.reshape(1, -1))
    return out


# speedup vs baseline: 40.7563x; 1.0009x over previous
import functools

import jax
import jax.numpy as jnp
from jax import lax
from jax.experimental import pallas as pl
from jax.experimental.pallas import tpu as pltpu
from jax.experimental.pallas import tpu_sc as plsc

N = 10000
E = 320000
D = 128
DE = 16
G = 256

NC = 2
NS = 16
NW = NC * NS

LT = E // 128
TPW = LT // NW
EPW = TPW * 128
XTRA = LT - TPW * NW
XBASE = TPW * NW

CH = 3
NCH = TPW // CH
ECH = CH * 128

NB = 16
SEGW = G * DE

PN = 40
NPN = N // PN


def kernel(x, edge_index, edge_attr, u, batch, W1, b1, g1, be1, W2, b2, g2, be2, W3, b3):
    ea_t = edge_attr.T

    mesh = plsc.VectorSubcoreMesh(core_axis_name="c", subcore_axis_name="s",
                                  num_cores=NC, num_subcores=NS)

    @functools.partial(
        pl.kernel,
        out_type=(
            jax.ShapeDtypeStruct((NC, G, D), jnp.float32),
            jax.ShapeDtypeStruct((NC, G, DE), jnp.float32),
        ),
        mesh=mesh,
        compiler_params=pltpu.CompilerParams(needs_layout_passes=False,
                                             skip_device_barrier=True),
        scratch_types=(
            pltpu.VMEM((N,), jnp.int32),
            pltpu.VMEM((EPW + 128,), jnp.int32),
            pltpu.VMEM((DE, ECH), jnp.float32),
            pltpu.VMEM((DE, ECH), jnp.float32),
            pltpu.VMEM((NB * SEGW,), jnp.float32),
            pltpu.VMEM((DE, DE), jnp.float32),
            pltpu.VMEM((PN, D), jnp.float32),
            pltpu.VMEM((PN, D), jnp.float32),
            pltpu.VMEM((2, PN), jnp.int32),
            pltpu.VMEM_SHARED((G, D), jnp.float32),
            pltpu.VMEM_SHARED((NB, SEGW), jnp.float32),
            pltpu.SemaphoreType.DMA,
            pltpu.SemaphoreType.DMA,
            pltpu.SemaphoreType.DMA,
            pltpu.SemaphoreType.DMA,
            pltpu.SemaphoreType.DMA,
        ),
    )
    def sc_agg(x_hbm, ei_hbm, eat_hbm, batch_hbm, node_out, edge_out,
               batch_v, cs_v, stage0_v, stage1_v, bank_v, out_v,
               x0_v, x1_v, nidx_v, acc_node, slab,
               sem_b, sem_e0, sem_e1, sem_n0, sem_n1):
        c = lax.axis_index("c")
        s = lax.axis_index("s")
        w = c * NS + s
        t0 = w * TPW
        e_base = t0 * 128

        zvec = jnp.zeros((16,), jnp.float32)
        lane_iota = lax.iota(jnp.int32, 16)
        lane_off = lane_iota * SEGW
        rots = [(lane_iota + d) & 15 for d in range(DE)]
        stages = [stage0_v, stage1_v]
        sems_e = [sem_e0, sem_e1]
        xbufs = [x0_v, x1_v]
        sems_n = [sem_n0, sem_n1]

        h_batch = pltpu.async_copy(batch_hbm, batch_v, sem_b)
        h_col = pltpu.async_copy(ei_hbm.at[1, pl.ds(e_base, EPW)],
                                 cs_v.at[pl.ds(0, EPW)], sem_b)

        @pl.when(w < XTRA)
        def _():
            pltpu.async_copy(ei_hbm.at[1, pl.ds((XBASE + w) * 128, 128)],
                             cs_v.at[pl.ds(EPW, 128)], sem_b)

        h_e = pltpu.async_copy(eat_hbm.at[:, pl.ds(e_base, ECH)], stage0_v,
                               sem_e0)

        for r in range(16):
            for k in range(D // 16):
                x0_v[r, pl.ds(k * 16, 16)] = zvec
        pltpu.sync_copy(x0_v.at[pl.ds(0, 16)], acc_node.at[pl.ds(s * 16, 16)])

        @pl.loop(0, NB * SEGW // 64, unroll=4)
        def _zero(i):
            for k in range(4):
                bank_v[pl.ds(i * 64 + k * 16, 16)] = zvec

        h_batch.wait()
        h_col.wait()

        @pl.when(w < XTRA)
        def _():
            pltpu.make_async_copy(ei_hbm.at[1, pl.ds(0, 128)],
                                  cs_v.at[pl.ds(EPW, 128)], sem_b).wait()

        @pl.loop(0, EPW // 16, unroll=4)
        def _gather(g):
            cv = cs_v[pl.ds(g * 16, 16)]
            cs_v[pl.ds(g * 16, 16)] = plsc.load_gather(batch_v, [cv])

        @pl.when(w < XTRA)
        def _():
            for k in range(128 // 16):
                cv = cs_v[pl.ds(EPW + k * 16, 16)]
                cs_v[pl.ds(EPW + k * 16, 16)] = plsc.load_gather(batch_v, [cv])

        plsc.subcore_barrier()

        def scatter_groups(buf, local_e0, ngroups):
            @pl.loop(0, ngroups)
            def _(g):
                seg16 = cs_v[pl.ds(local_e0 + g * 16, 16)]
                base = seg16 * DE + lane_off
                vals = [buf[d, pl.ds(g * 16, 16)] for d in range(DE)]
                idxs = [base + rots[d] for d in range(DE)]
                for d in range(DE):
                    plsc.addupdate_scatter(bank_v, [idxs[d]], vals[d])

        h_cur = h_e
        for cc in range(NCH):
            if cc + 1 < NCH:
                h_next = pltpu.async_copy(
                    eat_hbm.at[:, pl.ds(e_base + (cc + 1) * ECH, ECH)],
                    stages[(cc + 1) % 2], sems_e[(cc + 1) % 2])
            h_cur.wait()
            scatter_groups(stages[cc % 2], cc * ECH, ECH // 16)
            if cc + 1 < NCH:
                h_cur = h_next

        @pl.when(w < XTRA)
        def _():
            pltpu.sync_copy(eat_hbm.at[:, pl.ds((XBASE + w) * 128, 128)],
                            stage0_v.at[:, pl.ds(0, 128)])
            scatter_groups(stage0_v, EPW, 128 // 16)

        def prefetch(t):
            par = t % 2
            p = w + t * NW
            hi = pltpu.async_copy(batch_hbm.at[pl.ds(p * PN, PN)],
                                  nidx_v.at[par], sems_n[par])
            hx = pltpu.async_copy(x_hbm.at[pl.ds(p * PN, PN)], xbufs[par],
                                  sems_n[par])
            return hi, hx

        handles = [None, None]
        handles[0] = prefetch(0)
        for t in range(8):
            par = t % 2
            valid = t < 7
            if t + 1 < 7:
                handles[(t + 1) % 2] = prefetch(t + 1)
            elif t + 1 == 7:
                @pl.when(w + 7 * NW < NPN)
                def _():
                    par2 = (t + 1) % 2
                    pltpu.async_copy(batch_hbm.at[pl.ds((w + 7 * NW) * PN, PN)],
                                     nidx_v.at[par2], sems_n[par2])
                    pltpu.async_copy(x_hbm.at[pl.ds((w + 7 * NW) * PN, PN)],
                                     xbufs[par2], sems_n[par2])
            if valid:
                hi, hx = handles[par]
                hi.wait()
                hx.wait()
                pltpu.sync_copy(xbufs[par], acc_node.at[nidx_v.at[par]],
                                add=True)
            else:
                @pl.when(w + 7 * NW < NPN)
                def _():
                    pltpu.make_async_copy(batch_hbm.at[pl.ds(0, PN)],
                                          nidx_v.at[par], sems_n[par]).wait()
                    pltpu.make_async_copy(x_hbm.at[pl.ds(0, PN)], xbufs[par],
                                          sems_n[par]).wait()
                    pltpu.sync_copy(xbufs[par], acc_node.at[nidx_v.at[par]],
                                    add=True)

        @pl.loop(0, SEGW // 16)
        def _bankred(i):
            acc = bank_v[pl.ds(i * 16, 16)]
            for b in range(1, NB):
                idx = rots[b] + (b * SEGW + i * 16)
                acc = acc + plsc.load_gather(bank_v, [idx])
            bank_v[pl.ds(i * 16, 16)] = acc

        pltpu.sync_copy(bank_v.at[pl.ds(0, SEGW)], slab.at[s])
        plsc.subcore_barrier()
        pltpu.sync_copy(slab.at[:, pl.ds(s * G, G)],
                        stage0_v.at[:, pl.ds(0, G)])
        for i in range(DE):
            acc = zvec
            for b in range(NB):
                acc = acc + stage0_v[b, pl.ds(i * 16, 16)]
            out_v[i, :] = acc
        pltpu.sync_copy(out_v, edge_out.at[c, pl.ds(s * DE, DE), :])

        pltpu.sync_copy(acc_node.at[pl.ds(s * 16, 16)],
                        node_out.at[c, pl.ds(s * 16, 16)])

    node_p, edge_p = sc_agg(x, edge_index, ea_t, batch)

    def mlp_body(node_ref, edge_ref, W1_ref, b1_ref, g1_ref, be1_ref,
                 W2_ref, b2_ref, g2_ref, be2_ref, W3_ref, b3_ref, out_ref):
        na = node_ref[0] + node_ref[1]
        ea = edge_ref[0] + edge_ref[1]
        h = (jnp.dot(na, W1_ref[:D, :], preferred_element_type=jnp.float32)
             + jnp.dot(ea, W1_ref[D:, :], preferred_element_type=jnp.float32)
             + b1_ref[...])

        def act_bn(h, gamma, beta):
            h = jnp.where(h >= 0, h, 0.01 * h)
            mean = jnp.mean(h, axis=0, keepdims=True)
            var = jnp.mean((h - mean) ** 2, axis=0, keepdims=True)
            return (h - mean) / jnp.sqrt(var + 1e-5) * gamma + beta

        h = act_bn(h, g1_ref[...], be1_ref[...])
        h = jnp.dot(h, W2_ref[...], preferred_element_type=jnp.float32) + b2_ref[...]
        h = act_bn(h, g2_ref[...], be2_ref[...])
        out_ref[...] = (jnp.dot(h, W3_ref[...], preferred_element_type=jnp.float32)
                        + b3_ref[...])

    out = pl.pallas_call(
        mlp_body,
        out_shape=jax.ShapeDtypeStruct((G, D), jnp.float32),
    )(node_p, edge_p, W1, b1.reshape(1, -1), g1.reshape(1, -1),
      be1.reshape(1, -1), W2, b2.reshape(1, -1), g2.reshape(1, -1),
      be2.reshape(1, -1), W3, b3.reshape(1, -1))
    return out


# speedup vs baseline: 41.8934x; 1.0279x over previous
import functools

import jax
import jax.numpy as jnp
from jax import lax
from jax.experimental import pallas as pl
from jax.experimental.pallas import tpu as pltpu
from jax.experimental.pallas import tpu_sc as plsc

N = 10000
E = 320000
D = 128
DE = 16
G = 256

NC = 2
NS = 16
NW = NC * NS

LT = E // 128
TPW = LT // NW
EPW = TPW * 128
XTRA = LT - TPW * NW
XBASE = TPW * NW

CH = 3
NCH = TPW // CH
ECH = CH * 128

NB = 16
SEGW = G * DE

PN = 40
NPN = N // PN


def kernel(x, edge_index, edge_attr, u, batch, W1, b1, g1, be1, W2, b2, g2, be2, W3, b3):
    ea_t = edge_attr.T

    mesh = plsc.VectorSubcoreMesh(core_axis_name="c", subcore_axis_name="s",
                                  num_cores=NC, num_subcores=NS)

    @functools.partial(
        pl.kernel,
        out_type=(
            jax.ShapeDtypeStruct((NC, G, D), jnp.float32),
            jax.ShapeDtypeStruct((NC, G, DE), jnp.float32),
        ),
        mesh=mesh,
        compiler_params=pltpu.CompilerParams(needs_layout_passes=False),
        scratch_types=(
            pltpu.VMEM((N,), jnp.int32),
            pltpu.VMEM((EPW + 128,), jnp.int32),
            pltpu.VMEM((DE, ECH), jnp.float32),
            pltpu.VMEM((DE, ECH), jnp.float32),
            pltpu.VMEM((NB * SEGW,), jnp.float32),
            pltpu.VMEM((DE, DE), jnp.float32),
            pltpu.VMEM((PN, D), jnp.float32),
            pltpu.VMEM((PN, D), jnp.float32),
            pltpu.VMEM((2, PN), jnp.int32),
            pltpu.VMEM_SHARED((G, D), jnp.float32),
            pltpu.VMEM_SHARED((NB, SEGW), jnp.float32),
            pltpu.SemaphoreType.DMA,
            pltpu.SemaphoreType.DMA,
            pltpu.SemaphoreType.DMA,
            pltpu.SemaphoreType.DMA,
            pltpu.SemaphoreType.DMA,
        ),
    )
    def sc_agg(x_hbm, ei_hbm, eat_hbm, batch_hbm, node_out, edge_out,
               batch_v, cs_v, stage0_v, stage1_v, bank_v, out_v,
               x0_v, x1_v, nidx_v, acc_node, slab,
               sem_b, sem_e0, sem_e1, sem_n0, sem_n1):
        c = lax.axis_index("c")
        s = lax.axis_index("s")
        w = c * NS + s
        t0 = w * TPW
        e_base = t0 * 128

        zvec = jnp.zeros((16,), jnp.float32)
        lane_iota = lax.iota(jnp.int32, 16)
        lane_off = lane_iota * SEGW
        rots = [(lane_iota + d) & 15 for d in range(DE)]
        stages = [stage0_v, stage1_v]
        sems_e = [sem_e0, sem_e1]
        xbufs = [x0_v, x1_v]
        sems_n = [sem_n0, sem_n1]

        h_batch = pltpu.async_copy(batch_hbm, batch_v, sem_b)
        h_col = pltpu.async_copy(ei_hbm.at[1, pl.ds(e_base, EPW)],
                                 cs_v.at[pl.ds(0, EPW)], sem_b)

        @pl.when(w < XTRA)
        def _():
            pltpu.async_copy(ei_hbm.at[1, pl.ds((XBASE + w) * 128, 128)],
                             cs_v.at[pl.ds(EPW, 128)], sem_b)

        h_e = pltpu.async_copy(eat_hbm.at[:, pl.ds(e_base, ECH)], stage0_v,
                               sem_e0)

        for r in range(16):
            for k in range(D // 16):
                x0_v[r, pl.ds(k * 16, 16)] = zvec
        pltpu.sync_copy(x0_v.at[pl.ds(0, 16)], acc_node.at[pl.ds(s * 16, 16)])

        @pl.loop(0, NB * SEGW // 64, unroll=4)
        def _zero(i):
            for k in range(4):
                bank_v[pl.ds(i * 64 + k * 16, 16)] = zvec

        h_batch.wait()
        h_col.wait()

        @pl.when(w < XTRA)
        def _():
            pltpu.make_async_copy(ei_hbm.at[1, pl.ds(0, 128)],
                                  cs_v.at[pl.ds(EPW, 128)], sem_b).wait()

        plsc.subcore_barrier()

        def scatter_groups(buf, local_e0, ngroups):
            @pl.loop(0, ngroups)
            def _(g):
                col16 = cs_v[pl.ds(local_e0 + g * 16, 16)]
                seg16 = plsc.load_gather(batch_v, [col16])
                base = seg16 * DE + lane_off
                vals = [buf[d, pl.ds(g * 16, 16)] for d in range(DE)]
                idxs = [base + rots[d] for d in range(DE)]
                for d in range(DE):
                    plsc.addupdate_scatter(bank_v, [idxs[d]], vals[d])

        h_cur = h_e
        for cc in range(NCH):
            if cc + 1 < NCH:
                h_next = pltpu.async_copy(
                    eat_hbm.at[:, pl.ds(e_base + (cc + 1) * ECH, ECH)],
                    stages[(cc + 1) % 2], sems_e[(cc + 1) % 2])
            h_cur.wait()
            scatter_groups(stages[cc % 2], cc * ECH, ECH // 16)
            if cc + 1 < NCH:
                h_cur = h_next

        @pl.when(w < XTRA)
        def _():
            pltpu.sync_copy(eat_hbm.at[:, pl.ds((XBASE + w) * 128, 128)],
                            stage0_v.at[:, pl.ds(0, 128)])
            scatter_groups(stage0_v, EPW, 128 // 16)

        def prefetch(t):
            par = t % 2
            p = w + t * NW
            hi = pltpu.async_copy(batch_hbm.at[pl.ds(p * PN, PN)],
                                  nidx_v.at[par], sems_n[par])
            hx = pltpu.async_copy(x_hbm.at[pl.ds(p * PN, PN)], xbufs[par],
                                  sems_n[par])
            return hi, hx

        handles = [None, None]
        handles[0] = prefetch(0)
        for t in range(8):
            par = t % 2
            valid = t < 7
            if t + 1 < 7:
                handles[(t + 1) % 2] = prefetch(t + 1)
            elif t + 1 == 7:
                @pl.when(w + 7 * NW < NPN)
                def _():
                    par2 = (t + 1) % 2
                    pltpu.async_copy(batch_hbm.at[pl.ds((w + 7 * NW) * PN, PN)],
                                     nidx_v.at[par2], sems_n[par2])
                    pltpu.async_copy(x_hbm.at[pl.ds((w + 7 * NW) * PN, PN)],
                                     xbufs[par2], sems_n[par2])
            if valid:
                hi, hx = handles[par]
                hi.wait()
                hx.wait()
                pltpu.sync_copy(xbufs[par], acc_node.at[nidx_v.at[par]],
                                add=True)
            else:
                @pl.when(w + 7 * NW < NPN)
                def _():
                    pltpu.make_async_copy(batch_hbm.at[pl.ds(0, PN)],
                                          nidx_v.at[par], sems_n[par]).wait()
                    pltpu.make_async_copy(x_hbm.at[pl.ds(0, PN)], xbufs[par],
                                          sems_n[par]).wait()
                    pltpu.sync_copy(xbufs[par], acc_node.at[nidx_v.at[par]],
                                    add=True)

        @pl.loop(0, SEGW // 16)
        def _bankred(i):
            acc = bank_v[pl.ds(i * 16, 16)]
            for b in range(1, NB):
                idx = rots[b] + (b * SEGW + i * 16)
                acc = acc + plsc.load_gather(bank_v, [idx])
            bank_v[pl.ds(i * 16, 16)] = acc

        pltpu.sync_copy(bank_v.at[pl.ds(0, SEGW)], slab.at[s])
        plsc.subcore_barrier()
        pltpu.sync_copy(slab.at[:, pl.ds(s * G, G)],
                        stage0_v.at[:, pl.ds(0, G)])
        for i in range(DE):
            acc = zvec
            for b in range(NB):
                acc = acc + stage0_v[b, pl.ds(i * 16, 16)]
            out_v[i, :] = acc
        pltpu.sync_copy(out_v, edge_out.at[c, pl.ds(s * DE, DE), :])

        pltpu.sync_copy(acc_node.at[pl.ds(s * 16, 16)],
                        node_out.at[c, pl.ds(s * 16, 16)])

    node_p, edge_p = sc_agg(x, edge_index, ea_t, batch)

    def mlp_body(node_ref, edge_ref, W1_ref, b1_ref, g1_ref, be1_ref,
                 W2_ref, b2_ref, g2_ref, be2_ref, W3_ref, b3_ref, out_ref):
        na = node_ref[0] + node_ref[1]
        ea = edge_ref[0] + edge_ref[1]
        h = (jnp.dot(na, W1_ref[:D, :], preferred_element_type=jnp.float32)
             + jnp.dot(ea, W1_ref[D:, :], preferred_element_type=jnp.float32)
             + b1_ref[...])

        def act_bn(h, gamma, beta):
            h = jnp.where(h >= 0, h, 0.01 * h)
            mean = jnp.mean(h, axis=0, keepdims=True)
            var = jnp.mean((h - mean) ** 2, axis=0, keepdims=True)
            return (h - mean) / jnp.sqrt(var + 1e-5) * gamma + beta

        h = act_bn(h, g1_ref[...], be1_ref[...])
        h = jnp.dot(h, W2_ref[...], preferred_element_type=jnp.float32) + b2_ref[...]
        h = act_bn(h, g2_ref[...], be2_ref[...])
        out_ref[...] = (jnp.dot(h, W3_ref[...], preferred_element_type=jnp.float32)
                        + b3_ref[...])

    out = pl.pallas_call(
        mlp_body,
        out_shape=jax.ShapeDtypeStruct((G, D), jnp.float32),
    )(node_p, edge_p, W1, b1.reshape(1, -1), g1.reshape(1, -1),
      be1.reshape(1, -1), W2, b2.reshape(1, -1), g2.reshape(1, -1),
      be2.reshape(1, -1), W3, b3.reshape(1, -1))
    return out


# speedup vs baseline: 44.2646x; 1.0566x over previous
import functools

import jax
import jax.numpy as jnp
from jax import lax
from jax.experimental import pallas as pl
from jax.experimental.pallas import tpu as pltpu
from jax.experimental.pallas import tpu_sc as plsc

N = 10000
E = 320000
D = 128
DE = 16
G = 256

NC = 2
NS = 16
NW = NC * NS

LT = E // 128
TPW = LT // NW
EPW = TPW * 128
XTRA = LT - TPW * NW
XBASE = TPW * NW

CH = 3
NCH = TPW // CH
ECH = CH * 128

NB = 16
SEGW = G * DE

PN = 40
NPN = N // PN


def kernel(x, edge_index, edge_attr, u, batch, W1, b1, g1, be1, W2, b2, g2, be2, W3, b3):
    ea_t = edge_attr.T

    mesh = plsc.VectorSubcoreMesh(core_axis_name="c", subcore_axis_name="s",
                                  num_cores=NC, num_subcores=NS)

    @functools.partial(
        pl.kernel,
        out_type=(
            jax.ShapeDtypeStruct((NC, G, D), jnp.float32),
            jax.ShapeDtypeStruct((NC, G, DE), jnp.float32),
        ),
        mesh=mesh,
        compiler_params=pltpu.CompilerParams(needs_layout_passes=False),
        scratch_types=(
            pltpu.VMEM((N,), jnp.int32),
            pltpu.VMEM((EPW + 128,), jnp.int32),
            pltpu.VMEM((DE, ECH), jnp.float32),
            pltpu.VMEM((DE, ECH), jnp.float32),
            pltpu.VMEM((NB * SEGW,), jnp.float32),
            pltpu.VMEM((DE, DE), jnp.float32),
            pltpu.VMEM((PN, D), jnp.float32),
            pltpu.VMEM((PN, D), jnp.float32),
            pltpu.VMEM((2, PN), jnp.int32),
            pltpu.VMEM_SHARED((G, D), jnp.float32),
            pltpu.VMEM_SHARED((NB, SEGW), jnp.float32),
            pltpu.SemaphoreType.DMA,
            pltpu.SemaphoreType.DMA,
            pltpu.SemaphoreType.DMA,
            pltpu.SemaphoreType.DMA,
            pltpu.SemaphoreType.DMA,
        ),
    )
    def sc_agg(x_hbm, ei_hbm, eat_hbm, batch_hbm, node_out, edge_out,
               batch_v, cs_v, stage0_v, stage1_v, bank_v, out_v,
               x0_v, x1_v, nidx_v, acc_node, slab,
               sem_b, sem_e0, sem_e1, sem_n0, sem_n1):
        c = lax.axis_index("c")
        s = lax.axis_index("s")
        w = c * NS + s
        t0 = w * TPW
        e_base = t0 * 128

        zvec = jnp.zeros((16,), jnp.float32)
        lane_iota = lax.iota(jnp.int32, 16)
        lane_off = lane_iota * SEGW
        rots = [(lane_iota + d) & 15 for d in range(DE)]
        stages = [stage0_v, stage1_v]
        sems_e = [sem_e0, sem_e1]
        xbufs = [x0_v, x1_v]
        sems_n = [sem_n0, sem_n1]

        h_batch = pltpu.async_copy(batch_hbm, batch_v, sem_b)
        h_col = pltpu.async_copy(ei_hbm.at[1, pl.ds(e_base, EPW)],
                                 cs_v.at[pl.ds(0, EPW)], sem_b)

        @pl.when(w < XTRA)
        def _():
            pltpu.async_copy(ei_hbm.at[1, pl.ds((XBASE + w) * 128, 128)],
                             cs_v.at[pl.ds(EPW, 128)], sem_b)

        h_e = pltpu.async_copy(eat_hbm.at[:, pl.ds(e_base, ECH)], stage0_v,
                               sem_e0)

        for r in range(16):
            for k in range(D // 16):
                x0_v[r, pl.ds(k * 16, 16)] = zvec
        pltpu.sync_copy(x0_v.at[pl.ds(0, 16)], acc_node.at[pl.ds(s * 16, 16)])

        @pl.loop(0, NB * SEGW // 64, unroll=4)
        def _zero(i):
            for k in range(4):
                bank_v[pl.ds(i * 64 + k * 16, 16)] = zvec

        h_batch.wait()
        h_col.wait()

        @pl.when(w < XTRA)
        def _():
            pltpu.make_async_copy(ei_hbm.at[1, pl.ds(0, 128)],
                                  cs_v.at[pl.ds(EPW, 128)], sem_b).wait()

        plsc.subcore_barrier()

        def scatter_groups(buf, local_e0, ngroups):
            @pl.loop(0, ngroups)
            def _(g):
                col16 = cs_v[pl.ds(local_e0 + g * 16, 16)]
                seg16 = plsc.load_gather(batch_v, [col16])
                base = seg16 * DE + lane_off
                vals = [buf[d, pl.ds(g * 16, 16)] for d in range(DE)]
                idxs = [base + rots[d] for d in range(DE)]
                for d in range(DE):
                    plsc.addupdate_scatter(bank_v, [idxs[d]], vals[d])

        def prefetch(t):
            par = t % 2
            p = w + t * NW
            hi = pltpu.async_copy(batch_hbm.at[pl.ds(p * PN, PN)],
                                  nidx_v.at[par], sems_n[par])
            hx = pltpu.async_copy(x_hbm.at[pl.ds(p * PN, PN)], xbufs[par],
                                  sems_n[par])
            return hi, hx

        handles = [None, None]

        def node_piece(t):
            par = t % 2
            if t + 1 < 7:
                handles[(t + 1) % 2] = prefetch(t + 1)
            elif t + 1 == 7:
                @pl.when(w + 7 * NW < NPN)
                def _():
                    par2 = (t + 1) % 2
                    pltpu.async_copy(batch_hbm.at[pl.ds((w + 7 * NW) * PN, PN)],
                                     nidx_v.at[par2], sems_n[par2])
                    pltpu.async_copy(x_hbm.at[pl.ds((w + 7 * NW) * PN, PN)],
                                     xbufs[par2], sems_n[par2])
            if t < 7:
                hi, hx = handles[par]
                hi.wait()
                hx.wait()
                pltpu.sync_copy(xbufs[par], acc_node.at[nidx_v.at[par]],
                                add=True)
            else:
                @pl.when(w + 7 * NW < NPN)
                def _():
                    pltpu.make_async_copy(batch_hbm.at[pl.ds(0, PN)],
                                          nidx_v.at[par], sems_n[par]).wait()
                    pltpu.make_async_copy(x_hbm.at[pl.ds(0, PN)], xbufs[par],
                                          sems_n[par]).wait()
                    pltpu.sync_copy(xbufs[par], acc_node.at[nidx_v.at[par]],
                                    add=True)

        handles[0] = prefetch(0)
        h_cur = h_e
        nt = 0
        for cc in range(NCH):
            if cc + 1 < NCH:
                h_next = pltpu.async_copy(
                    eat_hbm.at[:, pl.ds(e_base + (cc + 1) * ECH, ECH)],
                    stages[(cc + 1) % 2], sems_e[(cc + 1) % 2])
            h_cur.wait()
            scatter_groups(stages[cc % 2], cc * ECH, ECH // 16)
            if cc % 3 == 1 and nt < 8:
                node_piece(nt)
                nt += 1
            if cc + 1 < NCH:
                h_cur = h_next

        @pl.when(w < XTRA)
        def _():
            pltpu.sync_copy(eat_hbm.at[:, pl.ds((XBASE + w) * 128, 128)],
                            stage0_v.at[:, pl.ds(0, 128)])
            scatter_groups(stage0_v, EPW, 128 // 16)

        @pl.loop(0, SEGW // 16, unroll=2)
        def _bankred(i):
            acc = bank_v[pl.ds(i * 16, 16)]
            for b in range(1, NB):
                idx = rots[b] + (b * SEGW + i * 16)
                acc = acc + plsc.load_gather(bank_v, [idx])
            bank_v[pl.ds(i * 16, 16)] = acc

        pltpu.sync_copy(bank_v.at[pl.ds(0, SEGW)], slab.at[s])
        plsc.subcore_barrier()
        pltpu.sync_copy(slab.at[:, pl.ds(s * G, G)],
                        stage0_v.at[:, pl.ds(0, G)])
        for i in range(DE):
            acc = zvec
            for b in range(NB):
                acc = acc + stage0_v[b, pl.ds(i * 16, 16)]
            out_v[i, :] = acc
        pltpu.sync_copy(out_v, edge_out.at[c, pl.ds(s * DE, DE), :])

        pltpu.sync_copy(acc_node.at[pl.ds(s * 16, 16)],
                        node_out.at[c, pl.ds(s * 16, 16)])

    node_p, edge_p = sc_agg(x, edge_index, ea_t, batch)

    def mlp_body(node_ref, edge_ref, W1_ref, b1_ref, g1_ref, be1_ref,
                 W2_ref, b2_ref, g2_ref, be2_ref, W3_ref, b3_ref, out_ref):
        na = node_ref[0] + node_ref[1]
        ea = edge_ref[0] + edge_ref[1]
        h = (jnp.dot(na, W1_ref[:D, :], preferred_element_type=jnp.float32)
             + jnp.dot(ea, W1_ref[D:, :], preferred_element_type=jnp.float32)
             + b1_ref[...])

        def act_bn(h, gamma, beta):
            h = jnp.where(h >= 0, h, 0.01 * h)
            mean = jnp.mean(h, axis=0, keepdims=True)
            var = jnp.mean((h - mean) ** 2, axis=0, keepdims=True)
            return (h - mean) / jnp.sqrt(var + 1e-5) * gamma + beta

        h = act_bn(h, g1_ref[...], be1_ref[...])
        h = jnp.dot(h, W2_ref[...], preferred_element_type=jnp.float32) + b2_ref[...]
        h = act_bn(h, g2_ref[...], be2_ref[...])
        out_ref[...] = (jnp.dot(h, W3_ref[...], preferred_element_type=jnp.float32)
                        + b3_ref[...])

    out = pl.pallas_call(
        mlp_body,
        out_shape=jax.ShapeDtypeStruct((G, D), jnp.float32),
    )(node_p, edge_p, W1, b1.reshape(1, -1), g1.reshape(1, -1),
      be1.reshape(1, -1), W2, b2.reshape(1, -1), g2.reshape(1, -1),
      be2.reshape(1, -1), W3, b3.reshape(1, -1))
    return out


# speedup vs baseline: 46.2275x; 1.0443x over previous
import functools

import jax
import jax.numpy as jnp
from jax import lax
from jax.experimental import pallas as pl
from jax.experimental.pallas import tpu as pltpu
from jax.experimental.pallas import tpu_sc as plsc

N = 10000
E = 320000
D = 128
DE = 16
G = 256

NC = 2
NS = 16
NW = NC * NS

LT = E // 128
TPW = LT // NW
EPW = TPW * 128
XTRA = LT - TPW * NW
XBASE = TPW * NW

CH = 4
NCH = TPW // CH
ECH = CH * 128

NB = 16
SEGW = G * DE

PN = 40
NPN = N // PN


def kernel(x, edge_index, edge_attr, u, batch, W1, b1, g1, be1, W2, b2, g2, be2, W3, b3):
    ea_t = edge_attr.T

    mesh = plsc.VectorSubcoreMesh(core_axis_name="c", subcore_axis_name="s",
                                  num_cores=NC, num_subcores=NS)

    @functools.partial(
        pl.kernel,
        out_type=(
            jax.ShapeDtypeStruct((NC, G, D), jnp.float32),
            jax.ShapeDtypeStruct((NC, G, DE), jnp.float32),
        ),
        mesh=mesh,
        compiler_params=pltpu.CompilerParams(needs_layout_passes=False),
        scratch_types=(
            pltpu.VMEM((N,), jnp.int32),
            pltpu.VMEM((EPW + 128,), jnp.int32),
            pltpu.VMEM((DE, ECH), jnp.float32),
            pltpu.VMEM((DE, ECH), jnp.float32),
            pltpu.VMEM((NB * SEGW,), jnp.float32),
            pltpu.VMEM((DE, DE), jnp.float32),
            pltpu.VMEM((PN, D), jnp.float32),
            pltpu.VMEM((PN, D), jnp.float32),
            pltpu.VMEM((2, PN), jnp.int32),
            pltpu.VMEM_SHARED((G, D), jnp.float32),
            pltpu.VMEM_SHARED((NB, SEGW), jnp.float32),
            pltpu.SemaphoreType.DMA,
            pltpu.SemaphoreType.DMA,
            pltpu.SemaphoreType.DMA,
            pltpu.SemaphoreType.DMA,
            pltpu.SemaphoreType.DMA,
        ),
    )
    def sc_agg(x_hbm, ei_hbm, eat_hbm, batch_hbm, node_out, edge_out,
               batch_v, cs_v, stage0_v, stage1_v, bank_v, out_v,
               x0_v, x1_v, nidx_v, acc_node, slab,
               sem_b, sem_e0, sem_e1, sem_n0, sem_n1):
        c = lax.axis_index("c")
        s = lax.axis_index("s")
        w = c * NS + s
        t0 = w * TPW
        e_base = t0 * 128

        zvec = jnp.zeros((16,), jnp.float32)
        lane_iota = lax.iota(jnp.int32, 16)
        lane_off = lane_iota * SEGW
        rots = [(lane_iota + d) & 15 for d in range(DE)]
        stages = [stage0_v, stage1_v]
        sems_e = [sem_e0, sem_e1]
        xbufs = [x0_v, x1_v]
        sems_n = [sem_n0, sem_n1]

        h_batch = pltpu.async_copy(batch_hbm, batch_v, sem_b)
        h_col = pltpu.async_copy(ei_hbm.at[1, pl.ds(e_base, EPW)],
                                 cs_v.at[pl.ds(0, EPW)], sem_b)

        @pl.when(w < XTRA)
        def _():
            pltpu.async_copy(ei_hbm.at[1, pl.ds((XBASE + w) * 128, 128)],
                             cs_v.at[pl.ds(EPW, 128)], sem_b)

        h_e = pltpu.async_copy(eat_hbm.at[:, pl.ds(e_base, ECH)], stage0_v,
                               sem_e0)

        for r in range(16):
            for k in range(D // 16):
                x0_v[r, pl.ds(k * 16, 16)] = zvec
        pltpu.sync_copy(x0_v.at[pl.ds(0, 16)], acc_node.at[pl.ds(s * 16, 16)])

        @pl.loop(0, NB * SEGW // 64, unroll=4)
        def _zero(i):
            for k in range(4):
                bank_v[pl.ds(i * 64 + k * 16, 16)] = zvec

        h_batch.wait()
        h_col.wait()

        @pl.when(w < XTRA)
        def _():
            pltpu.make_async_copy(ei_hbm.at[1, pl.ds(0, 128)],
                                  cs_v.at[pl.ds(EPW, 128)], sem_b).wait()

        plsc.subcore_barrier()

        def scatter_groups(buf, local_e0, ngroups):
            @pl.loop(0, ngroups)
            def _(g):
                col16 = cs_v[pl.ds(local_e0 + g * 16, 16)]
                seg16 = plsc.load_gather(batch_v, [col16])
                base = seg16 * DE + lane_off
                vals = [buf[d, pl.ds(g * 16, 16)] for d in range(DE)]
                idxs = [base + rots[d] for d in range(DE)]
                for d in range(DE):
                    plsc.addupdate_scatter(bank_v, [idxs[d]], vals[d])

        def prefetch(t):
            par = t % 2
            p = w + t * NW
            hi = pltpu.async_copy(batch_hbm.at[pl.ds(p * PN, PN)],
                                  nidx_v.at[par], sems_n[par])
            hx = pltpu.async_copy(x_hbm.at[pl.ds(p * PN, PN)], xbufs[par],
                                  sems_n[par])
            return hi, hx

        handles = [None, None]

        def node_piece(t):
            par = t % 2
            if t + 1 < 7:
                handles[(t + 1) % 2] = prefetch(t + 1)
            elif t + 1 == 7:
                @pl.when(w + 7 * NW < NPN)
                def _():
                    par2 = (t + 1) % 2
                    pltpu.async_copy(batch_hbm.at[pl.ds((w + 7 * NW) * PN, PN)],
                                     nidx_v.at[par2], sems_n[par2])
                    pltpu.async_copy(x_hbm.at[pl.ds((w + 7 * NW) * PN, PN)],
                                     xbufs[par2], sems_n[par2])
            if t < 7:
                hi, hx = handles[par]
                hi.wait()
                hx.wait()
                pltpu.sync_copy(xbufs[par], acc_node.at[nidx_v.at[par]],
                                add=True)
            else:
                @pl.when(w + 7 * NW < NPN)
                def _():
                    pltpu.make_async_copy(batch_hbm.at[pl.ds(0, PN)],
                                          nidx_v.at[par], sems_n[par]).wait()
                    pltpu.make_async_copy(x_hbm.at[pl.ds(0, PN)], xbufs[par],
                                          sems_n[par]).wait()
                    pltpu.sync_copy(xbufs[par], acc_node.at[nidx_v.at[par]],
                                    add=True)

        handles[0] = prefetch(0)
        h_cur = h_e
        nt = 0
        for cc in range(NCH):
            if cc + 1 < NCH:
                h_next = pltpu.async_copy(
                    eat_hbm.at[:, pl.ds(e_base + (cc + 1) * ECH, ECH)],
                    stages[(cc + 1) % 2], sems_e[(cc + 1) % 2])
            h_cur.wait()
            scatter_groups(stages[cc % 2], cc * ECH, ECH // 16)
            if cc % 3 == 1 and nt < 8:
                node_piece(nt)
                nt += 1
            if cc + 1 < NCH:
                h_cur = h_next

        if TPW - NCH * CH:
            rem_e = (TPW - NCH * CH) * 128
            pltpu.sync_copy(eat_hbm.at[:, pl.ds(e_base + NCH * ECH, rem_e)],
                            stage1_v.at[:, pl.ds(0, rem_e)])
            scatter_groups(stage1_v, NCH * ECH, rem_e // 16)

        @pl.when(w < XTRA)
        def _():
            pltpu.sync_copy(eat_hbm.at[:, pl.ds((XBASE + w) * 128, 128)],
                            stage0_v.at[:, pl.ds(0, 128)])
            scatter_groups(stage0_v, EPW, 128 // 16)

        @pl.loop(0, SEGW // 16, unroll=2)
        def _bankred(i):
            acc = bank_v[pl.ds(i * 16, 16)]
            for b in range(1, NB):
                idx = rots[b] + (b * SEGW + i * 16)
                acc = acc + plsc.load_gather(bank_v, [idx])
            bank_v[pl.ds(i * 16, 16)] = acc

        pltpu.sync_copy(bank_v.at[pl.ds(0, SEGW)], slab.at[s])
        plsc.subcore_barrier()
        pltpu.sync_copy(slab.at[:, pl.ds(s * G, G)],
                        stage0_v.at[:, pl.ds(0, G)])
        for i in range(DE):
            acc = zvec
            for b in range(NB):
                acc = acc + stage0_v[b, pl.ds(i * 16, 16)]
            out_v[i, :] = acc
        pltpu.sync_copy(out_v, edge_out.at[c, pl.ds(s * DE, DE), :])

        pltpu.sync_copy(acc_node.at[pl.ds(s * 16, 16)],
                        node_out.at[c, pl.ds(s * 16, 16)])

    node_p, edge_p = sc_agg(x, edge_index, ea_t, batch)

    def mlp_body(node_ref, edge_ref, W1_ref, b1_ref, g1_ref, be1_ref,
                 W2_ref, b2_ref, g2_ref, be2_ref, W3_ref, b3_ref, out_ref):
        na = node_ref[0] + node_ref[1]
        ea = edge_ref[0] + edge_ref[1]
        h = (jnp.dot(na, W1_ref[:D, :], preferred_element_type=jnp.float32)
             + jnp.dot(ea, W1_ref[D:, :], preferred_element_type=jnp.float32)
             + b1_ref[...])

        def act_bn(h, gamma, beta):
            h = jnp.where(h >= 0, h, 0.01 * h)
            mean = jnp.mean(h, axis=0, keepdims=True)
            var = jnp.mean((h - mean) ** 2, axis=0, keepdims=True)
            return (h - mean) / jnp.sqrt(var + 1e-5) * gamma + beta

        h = act_bn(h, g1_ref[...], be1_ref[...])
        h = jnp.dot(h, W2_ref[...], preferred_element_type=jnp.float32) + b2_ref[...]
        h = act_bn(h, g2_ref[...], be2_ref[...])
        out_ref[...] = (jnp.dot(h, W3_ref[...], preferred_element_type=jnp.float32)
                        + b3_ref[...])

    out = pl.pallas_call(
        mlp_body,
        out_shape=jax.ShapeDtypeStruct((G, D), jnp.float32),
    )(node_p, edge_p, W1, b1.reshape(1, -1), g1.reshape(1, -1),
      be1.reshape(1, -1), W2, b2.reshape(1, -1), g2.reshape(1, -1),
      be2.reshape(1, -1), W3, b3.reshape(1, -1))
    return out
